# Initial kernel scaffold; baseline (speedup 1.0000x reference)
import jax
import jax.numpy as jnp
from jax.experimental import pallas as pl


def kernel(x, table):
    raise NotImplementedError("write your pallas kernel here")


# speedup vs baseline: 2.1055x; 2.1055x over previous
import functools

import numpy as np
import jax
import jax.numpy as jnp
from jax import lax
from jax.experimental import pallas as pl
from jax.experimental.pallas import tpu as pltpu
from jax.experimental.pallas import tpu_sc as plsc

_MAXLEN = 200
_VOCAB = 100000
_DIM = 128
_BATCH = 4096
_CH = 128
_NW = 32
_TOTAL_ROWS = _BATCH * _MAXLEN
_TOTAL_CHUNKS = _TOTAL_ROWS // _CH
_CPW = _TOTAL_CHUNKS // _NW
_LANES = 16


def _pos_table() -> np.ndarray:
    position = np.arange(_MAXLEN)[:, np.newaxis]
    i = np.arange(_DIM)[np.newaxis, :]
    angles = 1 / np.power(10000, 2 * (i // 2) / np.float32(_DIM))
    angle_rads = position * angles
    angle_rads[:, 0::2] = np.sin(angle_rads[:, 0::2])
    angle_rads[:, 1::2] = np.cos(angle_rads[:, 1::2])
    return np.asarray(angle_rads, dtype=np.float32)


_POS2 = np.concatenate([_pos_table(), _pos_table()], axis=0)


def _body(idx_hbm, table_hbm, pos_hbm, out_hbm, idx_v, pos_v, rows_v, sem):
    nc = 2
    wid = lax.axis_index("s") * nc + lax.axis_index("c")
    cbase = wid * _CPW

    pltpu.sync_copy(idx_hbm.at[pl.ds(cbase, _CPW)], idx_v)
    pltpu.sync_copy(pos_hbm, pos_v)

    def step(g, carry):
        pltpu.async_copy(table_hbm.at[idx_v.at[g]], rows_v, sem).wait()
        l0 = (g * _CH) % _MAXLEN

        def add_row(r, c2):
            for c in range(_DIM // _LANES):
                sl = pl.ds(c * _LANES, _LANES)
                rows_v[r, sl] = rows_v[r, sl] + pos_v[l0 + r, sl]
            return c2

        lax.fori_loop(0, _CH, add_row, 0, unroll=2)
        pltpu.sync_copy(rows_v, out_hbm.at[pl.ds((cbase + g) * _CH, _CH)])
        return carry

    lax.fori_loop(0, _CPW, step, 0)


@functools.partial(
    pl.kernel,
    out_type=jax.ShapeDtypeStruct((_TOTAL_ROWS, _DIM), jnp.float32),
    mesh=plsc.VectorSubcoreMesh(core_axis_name="c", subcore_axis_name="s"),
    scratch_types=[
        pltpu.VMEM((_CPW, _CH), jnp.int32),
        pltpu.VMEM((2 * _MAXLEN, _DIM), jnp.float32),
        pltpu.VMEM((_CH, _DIM), jnp.float32),
        pltpu.SemaphoreType.DMA,
    ],
)
def _sc_embed(idx_hbm, table_hbm, pos_hbm, out_hbm, idx_v, pos_v, rows_v, sem):
    _body(idx_hbm, table_hbm, pos_hbm, out_hbm, idx_v, pos_v, rows_v, sem)


def kernel(x, table):
    idx2d = x.astype(jnp.int32).reshape(_TOTAL_CHUNKS, _CH)
    pos2 = jnp.asarray(_POS2)
    out = _sc_embed(idx2d, table, pos2)
    return out.reshape(_BATCH, _MAXLEN, _DIM)


# speedup vs baseline: 4.9596x; 2.3556x over previous
import functools

import numpy as np
import jax
import jax.numpy as jnp
from jax import lax
from jax.experimental import pallas as pl
from jax.experimental.pallas import tpu as pltpu
from jax.experimental.pallas import tpu_sc as plsc

_MAXLEN = 200
_VOCAB = 100000
_DIM = 128
_BATCH = 4096
_CH = 128
_NW = 32
_TOTAL_ROWS = _BATCH * _MAXLEN
_TOTAL_CHUNKS = _TOTAL_ROWS // _CH
_CPW = _TOTAL_CHUNKS // _NW
_LANES = 16


def _pos_table() -> np.ndarray:
    position = np.arange(_MAXLEN)[:, np.newaxis]
    i = np.arange(_DIM)[np.newaxis, :]
    angles = 1 / np.power(10000, 2 * (i // 2) / np.float32(_DIM))
    angle_rads = position * angles
    angle_rads[:, 0::2] = np.sin(angle_rads[:, 0::2])
    angle_rads[:, 1::2] = np.cos(angle_rads[:, 1::2])
    return np.asarray(angle_rads, dtype=np.float32)


_POS2 = np.concatenate([_pos_table(), _pos_table()], axis=0)


def _body(idx_hbm, table_hbm, pos_hbm, out_hbm, idx_v, pos_sh, rows_v, sem):
    nc = 2
    wid = lax.axis_index("s") * nc + lax.axis_index("c")
    cbase = wid * _CPW

    pltpu.sync_copy(idx_hbm.at[pl.ds(cbase, _CPW)], idx_v)

    @pl.when(lax.axis_index("s") == 0)
    def _():
        pltpu.sync_copy(pos_hbm, pos_sh)

    plsc.subcore_barrier()

    def step(g, carry):
        l0 = (g * _CH) % _MAXLEN
        pltpu.async_copy(pos_sh.at[pl.ds(l0, _CH)], rows_v, sem).wait()
        pltpu.async_copy(table_hbm.at[idx_v.at[g]], rows_v, sem, add=True).wait()
        pltpu.sync_copy(rows_v, out_hbm.at[pl.ds((cbase + g) * _CH, _CH)])
        return carry

    lax.fori_loop(0, _CPW, step, 0)


@functools.partial(
    pl.kernel,
    out_type=jax.ShapeDtypeStruct((_TOTAL_ROWS, _DIM), jnp.float32),
    mesh=plsc.VectorSubcoreMesh(core_axis_name="c", subcore_axis_name="s"),
    scratch_types=[
        pltpu.VMEM((_CPW, _CH), jnp.int32),
        pltpu.VMEM_SHARED((2 * _MAXLEN, _DIM), jnp.float32),
        pltpu.VMEM((_CH, _DIM), jnp.float32),
        pltpu.SemaphoreType.DMA,
    ],
)
def _sc_embed(idx_hbm, table_hbm, pos_hbm, out_hbm, idx_v, pos_sh, rows_v, sem):
    _body(idx_hbm, table_hbm, pos_hbm, out_hbm, idx_v, pos_sh, rows_v, sem)


def kernel(x, table):
    idx2d = x.astype(jnp.int32).reshape(_TOTAL_CHUNKS, _CH)
    pos2 = jnp.asarray(_POS2)
    out = _sc_embed(idx2d, table, pos2)
    return out.reshape(_BATCH, _MAXLEN, _DIM)


# speedup vs baseline: 7.5077x; 1.5138x over previous
import functools

import numpy as np
import jax
import jax.numpy as jnp
from jax import lax
from jax.experimental import pallas as pl
from jax.experimental.pallas import tpu as pltpu
from jax.experimental.pallas import tpu_sc as plsc

_MAXLEN = 200
_VOCAB = 100000
_DIM = 128
_BATCH = 4096
_CH = 128
_NW = 32
_NB = 4
_TOTAL_ROWS = _BATCH * _MAXLEN
_TOTAL_CHUNKS = _TOTAL_ROWS // _CH
_CPW = _TOTAL_CHUNKS // _NW


def _pos_table() -> np.ndarray:
    position = np.arange(_MAXLEN)[:, np.newaxis]
    i = np.arange(_DIM)[np.newaxis, :]
    angles = 1 / np.power(10000, 2 * (i // 2) / np.float32(_DIM))
    angle_rads = position * angles
    angle_rads[:, 0::2] = np.sin(angle_rads[:, 0::2])
    angle_rads[:, 1::2] = np.cos(angle_rads[:, 1::2])
    return np.asarray(angle_rads, dtype=np.float32)


_POS2 = np.concatenate([_pos_table(), _pos_table()], axis=0)


def _body(idx_hbm, table_hbm, pos_hbm, out_hbm, idx_v, pos_sh, rows_v, sp, sg, ss):
    nc = 2
    wid = lax.axis_index("s") * nc + lax.axis_index("c")
    cbase = wid * _CPW

    pltpu.sync_copy(idx_hbm.at[pl.ds(cbase, _CPW)], idx_v)

    @pl.when(lax.axis_index("s") == 0)
    def _():
        pltpu.sync_copy(pos_hbm, pos_sh)

    plsc.subcore_barrier()

    def prefill_start(g, b):
        l0 = (g * _CH) % _MAXLEN
        pltpu.async_copy(pos_sh.at[pl.ds(l0, _CH)], rows_v.at[b], sp[b])

    def prefill_wait(b):
        pltpu.make_async_copy(pos_sh.at[pl.ds(0, _CH)], rows_v.at[b], sp[b]).wait()

    def gather_start(g, b):
        return pltpu.async_copy(table_hbm.at[idx_v.at[g]], rows_v.at[b], sg[b],
                                add=True)

    def store_start(g, b):
        pltpu.async_copy(rows_v.at[b], out_hbm.at[pl.ds((cbase + g) * _CH, _CH)],
                         ss[b])

    def store_wait(b):
        pltpu.make_async_copy(rows_v.at[b], out_hbm.at[pl.ds(0, _CH)], ss[b]).wait()

    prefill_start(0, 0)
    prefill_start(1, 1)
    for g in (0, 1):
        prefill_wait(g)
        d = gather_start(g, g)
        prefill_start(g + 2, g + 2)
        d.wait()
        store_start(g, g)

    def group(i, carry):
        g0 = 2 + 4 * i
        for j in range(4):
            g = g0 + j
            b = (2 + j) % _NB
            prefill_wait(b)
            d = gather_start(g, b)
            store_wait(j)
            prefill_start(g + 2, j)
            d.wait()
            store_start(g, b)
        return carry

    lax.fori_loop(0, (_CPW - 4) // 4, group, 0)

    for g, b in ((_CPW - 2, 2), (_CPW - 1, 3)):
        prefill_wait(b)
        d = gather_start(g, b)
        d.wait()
        store_start(g, b)
    for b in range(_NB):
        store_wait(b)


@functools.partial(
    pl.kernel,
    out_type=jax.ShapeDtypeStruct((_TOTAL_ROWS, _DIM), jnp.float32),
    mesh=plsc.VectorSubcoreMesh(core_axis_name="c", subcore_axis_name="s"),
    scratch_types=[
        pltpu.VMEM((_CPW, _CH), jnp.int32),
        pltpu.VMEM_SHARED((2 * _MAXLEN, _DIM), jnp.float32),
        pltpu.VMEM((_NB, _CH, _DIM), jnp.float32),
        [pltpu.SemaphoreType.DMA] * _NB,
        [pltpu.SemaphoreType.DMA] * _NB,
        [pltpu.SemaphoreType.DMA] * _NB,
    ],
)
def _sc_embed(idx_hbm, table_hbm, pos_hbm, out_hbm, idx_v, pos_sh, rows_v,
              sp, sg, ss):
    _body(idx_hbm, table_hbm, pos_hbm, out_hbm, idx_v, pos_sh, rows_v, sp, sg, ss)


def kernel(x, table):
    idx2d = x.astype(jnp.int32).reshape(_TOTAL_CHUNKS, _CH)
    pos2 = jnp.asarray(_POS2)
    out = _sc_embed(idx2d, table, pos2)
    return out.reshape(_BATCH, _MAXLEN, _DIM)


# speedup vs baseline: 8.9450x; 1.1914x over previous
import functools

import numpy as np
import jax
import jax.numpy as jnp
from jax import lax
from jax.experimental import pallas as pl
from jax.experimental.pallas import tpu as pltpu
from jax.experimental.pallas import tpu_sc as plsc

_MAXLEN = 200
_VOCAB = 100000
_DIM = 128
_BATCH = 4096
_CH = 128
_NW = 32
_NB = 5
_TOTAL_ROWS = _BATCH * _MAXLEN
_TOTAL_CHUNKS = _TOTAL_ROWS // _CH
_CPW = _TOTAL_CHUNKS // _NW


def _pos_table() -> np.ndarray:
    position = np.arange(_MAXLEN)[:, np.newaxis]
    i = np.arange(_DIM)[np.newaxis, :]
    angles = 1 / np.power(10000, 2 * (i // 2) / np.float32(_DIM))
    angle_rads = position * angles
    angle_rads[:, 0::2] = np.sin(angle_rads[:, 0::2])
    angle_rads[:, 1::2] = np.cos(angle_rads[:, 1::2])
    return np.asarray(angle_rads, dtype=np.float32)


_POS2 = np.concatenate([_pos_table(), _pos_table()], axis=0)


def _body(idx_hbm, table_hbm, pos_hbm, out_hbm, idx_v, pos_sh, rows_v, sp, sg, ss):
    nc = 2
    wid = lax.axis_index("s") * nc + lax.axis_index("c")
    cbase = wid * _CPW

    pltpu.sync_copy(idx_hbm.at[pl.ds(cbase, _CPW)], idx_v)

    @pl.when(lax.axis_index("s") == 0)
    def _():
        pltpu.sync_copy(pos_hbm, pos_sh)

    plsc.subcore_barrier()

    def prefill_start(g, b):
        l0 = (g * _CH) % _MAXLEN
        pltpu.async_copy(pos_sh.at[pl.ds(l0, _CH)], rows_v.at[b], sp[b])

    def prefill_wait(b):
        pltpu.make_async_copy(pos_sh.at[pl.ds(0, _CH)], rows_v.at[b], sp[b]).wait()

    def gather_start(g, b):
        pltpu.async_copy(table_hbm.at[idx_v.at[g]], rows_v.at[b], sg[b],
                         add=True)

    def gather_wait(g, b):
        pltpu.make_async_copy(table_hbm.at[idx_v.at[g]], rows_v.at[b],
                              sg[b]).wait()

    def store_start(g, b):
        pltpu.async_copy(rows_v.at[b], out_hbm.at[pl.ds((cbase + g) * _CH, _CH)],
                         ss[b])

    def store_wait(b):
        pltpu.make_async_copy(rows_v.at[b], out_hbm.at[pl.ds(0, _CH)], ss[b]).wait()

    def chunk_step(g, b, with_store_wait=True):
        bp1 = (b + 1) % _NB
        bp2 = (b + 2) % _NB
        prefill_wait(bp1)
        gather_start(g + 1, bp1)
        gather_wait(g, b)
        store_start(g, b)
        if with_store_wait:
            store_wait(bp2)
        prefill_start(g + 2, bp2)

    prefill_start(0, 0)
    prefill_start(1, 1)
    prefill_wait(0)
    gather_start(0, 0)
    for g in (0, 1, 2):
        chunk_step(g, g, with_store_wait=False)

    def group(i, carry):
        g0 = 3 + 5 * i
        for j in range(5):
            chunk_step(g0 + j, (3 + j) % _NB)
        return carry

    lax.fori_loop(0, (_CPW - 5) // 5, group, 0)

    prefill_wait(4)
    gather_start(_CPW - 1, 4)
    gather_wait(_CPW - 2, 3)
    store_start(_CPW - 2, 3)
    store_wait(0)
    gather_wait(_CPW - 1, 4)
    store_start(_CPW - 1, 4)
    for b in (1, 2, 3, 4):
        store_wait(b)


@functools.partial(
    pl.kernel,
    out_type=jax.ShapeDtypeStruct((_TOTAL_ROWS, _DIM), jnp.float32),
    mesh=plsc.VectorSubcoreMesh(core_axis_name="c", subcore_axis_name="s"),
    scratch_types=[
        pltpu.VMEM((_CPW, _CH), jnp.int32),
        pltpu.VMEM_SHARED((2 * _MAXLEN, _DIM), jnp.float32),
        pltpu.VMEM((_NB, _CH, _DIM), jnp.float32),
        [pltpu.SemaphoreType.DMA] * _NB,
        [pltpu.SemaphoreType.DMA] * _NB,
        [pltpu.SemaphoreType.DMA] * _NB,
    ],
)
def _sc_embed(idx_hbm, table_hbm, pos_hbm, out_hbm, idx_v, pos_sh, rows_v,
              sp, sg, ss):
    _body(idx_hbm, table_hbm, pos_hbm, out_hbm, idx_v, pos_sh, rows_v, sp, sg, ss)


def kernel(x, table):
    idx2d = x.astype(jnp.int32).reshape(_TOTAL_CHUNKS, _CH)
    pos2 = jnp.asarray(_POS2)
    out = _sc_embed(idx2d, table, pos2)
    return out.reshape(_BATCH, _MAXLEN, _DIM)
